# baseline (speedup 1.0000x reference)
import numpy as np
import jax
import jax.numpy as jnp
from jax.experimental import pallas as pl

_FLOAT_MIN = -3.4e+38
_NE = 32
_ED = 128
_HID = 256
_CB = 256

_K1 = np.array([
    0x6d3e048f, 0x03d7b32d, 0x92fb20ea, 0xbad56946, 0xb013aee3, 0xa4d91a96,
    0xa506c508, 0x97d0552f, 0x3c999167, 0x4349448b, 0x3df0e1d3, 0x0eb4f3d8,
    0xb99c0582, 0xd181e6dd, 0x969d2d83, 0x3b1a4151, 0x058d3668, 0xa1128e08,
    0x38eb112d, 0xf7d22e13, 0x85daa268, 0x1e858c9e, 0x988b7618, 0xce913f97,
    0x69ac4320, 0xfda520b2, 0x8b3aa7c9, 0x21f80071, 0x846a0583, 0x339d7f6d,
    0x3b49834b, 0x660fd86e], dtype=np.uint32)
_K2 = np.array([
    0x1022172d, 0xadd083f4, 0x0f38d913, 0x354ba891, 0xc34eddf6, 0x3122544e,
    0xb6207291, 0x51bf719f, 0x8e776fea, 0x92d8bf3b, 0x645d7be2, 0xd1c5d1a8,
    0x549a95c6, 0x0db060e5, 0xecddd059, 0xea9246f9, 0xb396635b, 0x37234531,
    0x2a0b6421, 0x30c06e4a, 0xe24b3e2f, 0x00fb5046, 0x9e33c7e5, 0x6d300779,
    0x0e303598, 0x1eecb036, 0x9fe12574, 0x6a19312b, 0x1f2559e8, 0xc3ad5548,
    0xfac03e31, 0x1366dbec], dtype=np.uint32)

_TINY = np.float32(np.finfo(np.float32).tiny)
_SCALE = np.float32(np.float32(1.0) - _TINY)
_ROT_A = (13, 15, 26, 6)
_ROT_B = (17, 29, 16, 24)


def _rounds(x0, x1, rots):
    for r in rots:
        x0 = x0 + x1
        x1 = jnp.bitwise_xor(x0, (x1 << r) | (x1 >> (32 - r)))
    return x0, x1


def _gumbel(p, idx):
    ks0 = np.uint32(_K1[p])
    ks1 = np.uint32(_K2[p])
    ks2 = np.uint32(ks0 ^ ks1 ^ np.uint32(0x1BD11BDA))
    one = np.uint32(1)
    x0 = jnp.full(idx.shape, ks0, jnp.uint32)
    x1 = idx + ks1
    x0, x1 = _rounds(x0, x1, _ROT_A)
    x0, x1 = x0 + ks1, x1 + np.uint32(ks2 + one)
    x0, x1 = _rounds(x0, x1, _ROT_B)
    x0, x1 = x0 + ks2, x1 + np.uint32(ks0 + np.uint32(2))
    x0, x1 = _rounds(x0, x1, _ROT_A)
    x0, x1 = x0 + ks0, x1 + np.uint32(ks1 + np.uint32(3))
    x0, x1 = _rounds(x0, x1, _ROT_B)
    x0, x1 = x0 + ks1, x1 + np.uint32(ks2 + np.uint32(4))
    x0, x1 = _rounds(x0, x1, _ROT_A)
    x0, x1 = x0 + ks2, x1 + np.uint32(ks0 + np.uint32(5))
    bits = jnp.bitwise_xor(x0, x1)
    fb = (bits >> 9) | np.uint32(0x3F800000)
    f = jax.lax.bitcast_convert_type(fb, jnp.float32) - np.float32(1.0)
    u = jnp.maximum(_TINY, f * _SCALE + _TINY)
    return -jnp.log(-jnp.log(u))


def _fused_kernel(feats_ref, W1_ref, b1_ref, W2_ref, b2_ref, perm_ref, inv_ref):
    cb = feats_ref.shape[0]
    A = feats_ref[...].reshape(cb * _NE, _ED)
    h = jnp.dot(A, W1_ref[...], preferred_element_type=jnp.float32) + b1_ref[...]
    h = jnp.maximum(h, 0.0)
    L2 = jnp.dot(h, W2_ref[...], preferred_element_type=jnp.float32) + b2_ref[...]
    Lt = jnp.transpose(L2)
    L3 = Lt.reshape(_NE, cb, _NE)
    LT3 = jnp.swapaxes(L3, 1, 2)

    c0 = pl.program_id(0) * cb
    iota_c = jax.lax.broadcasted_iota(jnp.int32, (_NE, cb), 1)
    iota_e = jax.lax.broadcasted_iota(jnp.int32, (_NE, cb), 0)
    idx = ((c0 + iota_c) * _NE + iota_e).astype(jnp.uint32)

    maskneg = jnp.zeros((_NE, cb), jnp.float32)
    onehots = []
    for p in range(_NE):
        x = (LT3[p] + maskneg) + _gumbel(p, idx)
        m = jnp.max(x, axis=0, keepdims=True)
        ex = jnp.exp(x - m)
        s = jnp.sum(ex, axis=0, keepdims=True)
        r = ex / s
        rmax = jnp.max(r, axis=0, keepdims=True)
        sel = jnp.min(jnp.where(r == rmax, iota_e, _NE), axis=0, keepdims=True)
        oh = (iota_e == sel).astype(jnp.float32)
        maskneg = maskneg + oh * _FLOAT_MIN
        onehots.append(oh)
    OT = jnp.stack(onehots, axis=0)
    P1 = jnp.swapaxes(OT, 1, 2)
    perm = jnp.swapaxes(P1, 0, 1)
    nb = perm_ref.shape[0]
    perm_ref[...] = perm.reshape(nb, cb // nb, _NE, _NE)
    inv_ref[...] = jnp.swapaxes(perm, 1, 2).reshape(nb, cb // nb, _NE, _NE)


def kernel(ally_feats, enemy_feats, W1, b1, W2, b2,
           is_sample_action, is_target_net, t_env):
    del ally_feats, is_sample_action, is_target_net, t_env
    bs, na, ne, ed = enemy_feats.shape
    cells = bs * na
    feats = enemy_feats.reshape(cells, ne, ed)

    grid = (cells // _CB,)
    outT = pl.pallas_call(
        _fused_kernel,
        grid=grid,
        in_specs=[
            pl.BlockSpec((_CB, ne, ed), lambda i: (i, 0, 0)),
            pl.BlockSpec((ed, _HID), lambda i: (0, 0)),
            pl.BlockSpec((1, _HID), lambda i: (0, 0)),
            pl.BlockSpec((_HID, ne), lambda i: (0, 0)),
            pl.BlockSpec((1, ne), lambda i: (0, 0)),
        ],
        out_specs=[
            pl.BlockSpec((_CB // na, na, ne, ne), lambda i: (i, 0, 0, 0)),
            pl.BlockSpec((_CB // na, na, ne, ne), lambda i: (i, 0, 0, 0)),
        ],
        out_shape=[
            jax.ShapeDtypeStruct((bs, na, ne, ne), jnp.float32),
            jax.ShapeDtypeStruct((bs, na, ne, ne), jnp.float32),
        ],
    )(feats, W1, b1.reshape(1, _HID), W2, b2.reshape(1, ne))
    return outT


# speedup vs baseline: 1.1513x; 1.1513x over previous
import numpy as np
import jax
import jax.numpy as jnp
from jax.experimental import pallas as pl

_FLOAT_MIN = -3.4e+38
_NE = 32
_ED = 128
_HID = 256
_CB = 128

_K1 = np.array([
    0x6d3e048f, 0x03d7b32d, 0x92fb20ea, 0xbad56946, 0xb013aee3, 0xa4d91a96,
    0xa506c508, 0x97d0552f, 0x3c999167, 0x4349448b, 0x3df0e1d3, 0x0eb4f3d8,
    0xb99c0582, 0xd181e6dd, 0x969d2d83, 0x3b1a4151, 0x058d3668, 0xa1128e08,
    0x38eb112d, 0xf7d22e13, 0x85daa268, 0x1e858c9e, 0x988b7618, 0xce913f97,
    0x69ac4320, 0xfda520b2, 0x8b3aa7c9, 0x21f80071, 0x846a0583, 0x339d7f6d,
    0x3b49834b, 0x660fd86e], dtype=np.uint32)
_K2 = np.array([
    0x1022172d, 0xadd083f4, 0x0f38d913, 0x354ba891, 0xc34eddf6, 0x3122544e,
    0xb6207291, 0x51bf719f, 0x8e776fea, 0x92d8bf3b, 0x645d7be2, 0xd1c5d1a8,
    0x549a95c6, 0x0db060e5, 0xecddd059, 0xea9246f9, 0xb396635b, 0x37234531,
    0x2a0b6421, 0x30c06e4a, 0xe24b3e2f, 0x00fb5046, 0x9e33c7e5, 0x6d300779,
    0x0e303598, 0x1eecb036, 0x9fe12574, 0x6a19312b, 0x1f2559e8, 0xc3ad5548,
    0xfac03e31, 0x1366dbec], dtype=np.uint32)

_TINY = np.float32(np.finfo(np.float32).tiny)
_SCALE = np.float32(np.float32(1.0) - _TINY)
_ROT_A = (13, 15, 26, 6)
_ROT_B = (17, 29, 16, 24)


def _rounds(x0, x1, rots):
    for r in rots:
        x0 = x0 + x1
        x1 = jnp.bitwise_xor(x0, (x1 << r) | (x1 >> (32 - r)))
    return x0, x1


def _gumbel(p, idx):
    ks0 = np.uint32(_K1[p])
    ks1 = np.uint32(_K2[p])
    ks2 = np.uint32(ks0 ^ ks1 ^ np.uint32(0x1BD11BDA))
    one = np.uint32(1)
    x0 = jnp.full(idx.shape, ks0, jnp.uint32)
    x1 = idx + ks1
    x0, x1 = _rounds(x0, x1, _ROT_A)
    x0, x1 = x0 + ks1, x1 + np.uint32(ks2 + one)
    x0, x1 = _rounds(x0, x1, _ROT_B)
    x0, x1 = x0 + ks2, x1 + np.uint32(ks0 + np.uint32(2))
    x0, x1 = _rounds(x0, x1, _ROT_A)
    x0, x1 = x0 + ks0, x1 + np.uint32(ks1 + np.uint32(3))
    x0, x1 = _rounds(x0, x1, _ROT_B)
    x0, x1 = x0 + ks1, x1 + np.uint32(ks2 + np.uint32(4))
    x0, x1 = _rounds(x0, x1, _ROT_A)
    x0, x1 = x0 + ks2, x1 + np.uint32(ks0 + np.uint32(5))
    bits = jnp.bitwise_xor(x0, x1)
    fb = (bits >> 9) | np.uint32(0x3F800000)
    f = jax.lax.bitcast_convert_type(fb, jnp.float32) - np.float32(1.0)
    u = jnp.maximum(_TINY, f * _SCALE + _TINY)
    return -jnp.log(-jnp.log(u))


def _fused_kernel(feats_ref, W1_ref, b1_ref, W2_ref, b2_ref, perm_ref, inv_ref):
    cb = feats_ref.shape[0]
    A = feats_ref[...].reshape(cb * _NE, _ED)
    h = jnp.dot(A, W1_ref[...], preferred_element_type=jnp.float32) + b1_ref[...]
    h = jnp.maximum(h, 0.0)
    L2 = jnp.dot(h, W2_ref[...], preferred_element_type=jnp.float32) + b2_ref[...]
    Lt = jnp.transpose(L2)
    L3 = Lt.reshape(_NE, cb, _NE)
    LT3 = jnp.swapaxes(L3, 1, 2)

    c0 = pl.program_id(0) * cb
    iota_c = jax.lax.broadcasted_iota(jnp.int32, (_NE, cb), 1)
    iota_e = jax.lax.broadcasted_iota(jnp.int32, (_NE, cb), 0)
    idx = ((c0 + iota_c) * _NE + iota_e).astype(jnp.uint32)

    maskneg = jnp.zeros((_NE, cb), jnp.float32)
    onehots = []
    for p in range(_NE):
        x = (LT3[p] + maskneg) + _gumbel(p, idx)
        m = jnp.max(x, axis=0, keepdims=True)
        ex = jnp.exp(x - m)
        s = jnp.sum(ex, axis=0, keepdims=True)
        r = ex / s
        rmax = jnp.max(r, axis=0, keepdims=True)
        sel = jnp.min(jnp.where(r == rmax, iota_e, _NE), axis=0, keepdims=True)
        oh = (iota_e == sel).astype(jnp.float32)
        maskneg = maskneg + oh * _FLOAT_MIN
        onehots.append(oh)
    OT = jnp.stack(onehots, axis=0)
    P1 = jnp.swapaxes(OT, 1, 2)
    perm = jnp.swapaxes(P1, 0, 1)
    perm_ref[...] = perm
    inv_ref[...] = jnp.swapaxes(perm, 1, 2)


def kernel(ally_feats, enemy_feats, W1, b1, W2, b2,
           is_sample_action, is_target_net, t_env):
    del ally_feats, is_sample_action, is_target_net, t_env
    bs, na, ne, ed = enemy_feats.shape
    cells = bs * na
    feats = enemy_feats.reshape(cells, ne, ed)

    grid = (cells // _CB,)
    outT = pl.pallas_call(
        _fused_kernel,
        grid=grid,
        in_specs=[
            pl.BlockSpec((_CB, ne, ed), lambda i: (i, 0, 0)),
            pl.BlockSpec((ed, _HID), lambda i: (0, 0)),
            pl.BlockSpec((1, _HID), lambda i: (0, 0)),
            pl.BlockSpec((_HID, ne), lambda i: (0, 0)),
            pl.BlockSpec((1, ne), lambda i: (0, 0)),
        ],
        out_specs=[
            pl.BlockSpec((_CB, ne, ne), lambda i: (i, 0, 0)),
            pl.BlockSpec((_CB, ne, ne), lambda i: (i, 0, 0)),
        ],
        out_shape=[
            jax.ShapeDtypeStruct((cells, ne, ne), jnp.float32),
            jax.ShapeDtypeStruct((cells, ne, ne), jnp.float32),
        ],
    )(feats, W1, b1.reshape(1, _HID), W2, b2.reshape(1, ne))
    perm, inv = outT
    return (perm.reshape(bs, na, ne, ne), inv.reshape(bs, na, ne, ne))
